# Initial kernel scaffold; baseline (speedup 1.0000x reference)
import jax
import jax.numpy as jnp
from jax.experimental import pallas as pl


def kernel(x, edge_index, W1, b1, Ws, bs, W2, b2):
    raise NotImplementedError("write your pallas kernel here")


# speedup vs baseline: 47.2609x; 47.2609x over previous
import functools

import jax
import jax.numpy as jnp
from jax import lax
from jax.experimental import pallas as pl
from jax.experimental.pallas import tpu as pltpu
from jax.experimental.pallas import tpu_sc as plsc

N = 10000
E = 320000
D = 128
H = 16
C = 10
K = 5000

NC = 2
NS = 16
NW = NC * NS

BATCH = 128
NB = 80
EPW = NB * BATCH
EPAD = NW * EPW
PADROWS = 64
NPAD = 10240
SL = NPAD // NS
KPAD = 5120

_mesh = plsc.VectorSubcoreMesh(core_axis_name="c", subcore_axis_name="s")


def _make_agg(hdim, stage_table):
  tshape = (NPAD, hdim) if hdim > 1 else (NPAD,)
  oshape = (NC,) + tshape
  bufshape = (BATCH, hdim) if hdim > 1 else (BATCH,)
  zshape = (SL, hdim) if hdim > 1 else (SL,)
  scratch = [
      pltpu.VMEM((NB, BATCH), jnp.int32),
      pltpu.VMEM((NB, BATCH), jnp.int32),
      pltpu.VMEM(bufshape, jnp.float32),
      pltpu.VMEM(zshape, jnp.float32),
      pltpu.VMEM_SHARED(tshape, jnp.float32),
  ]
  if stage_table:
    scratch.append(pltpu.VMEM_SHARED(tshape, jnp.float32))

  @functools.partial(
      pl.kernel,
      out_type=jax.ShapeDtypeStruct(oshape, jnp.float32),
      mesh=_mesh,
      scratch_types=tuple(scratch),
      compiler_params=pltpu.CompilerParams(use_tc_tiling_on_sc=False, needs_layout_passes=False),
  )
  def agg(table_hbm, r_hbm, c_hbm, out_hbm, ridx, cidx, buf, zbuf, acc,
          *maybe_tab):
    cid = lax.axis_index("c")
    sid = lax.axis_index("s")
    wid = sid * NC + cid
    sl = pl.ds(sid * SL, SL)
    pltpu.sync_copy(r_hbm.at[wid], ridx)
    pltpu.sync_copy(c_hbm.at[wid], cidx)
    if stage_table:
      tab = maybe_tab[0]
      pltpu.sync_copy(table_hbm.at[sl], tab.at[sl])
    else:
      tab = table_hbm
    if hdim > 1:
      def zb(i, _):
        zbuf[i, :] = jnp.zeros((16,), jnp.float32)
        return 0
      lax.fori_loop(0, SL, zb, 0)
    else:
      def zb(i, _):
        zbuf[pl.ds(i * 16, 16)] = jnp.zeros((16,), jnp.float32)
        return 0
      lax.fori_loop(0, SL // 16, zb, 0)
    pltpu.sync_copy(zbuf, acc.at[sl])
    plsc.subcore_barrier()
    for b in range(NB):
      pltpu.sync_copy(tab.at[ridx.at[b]], buf)
      pltpu.sync_copy(buf, acc.at[cidx.at[b]], add=True)
    plsc.subcore_barrier()
    pltpu.sync_copy(acc.at[sl], out_hbm.at[cid, sl])

  return agg


_agg16 = _make_agg(H, stage_table=False)
_agg1 = _make_agg(1, stage_table=True)


_NBT = SL // BATCH


@functools.partial(
    pl.kernel,
    out_type=(
        jax.ShapeDtypeStruct((8, _NBT, BATCH), jnp.int32),
        jax.ShapeDtypeStruct((NPAD,), jnp.float32),
    ),
    mesh=_mesh,
    scratch_types=(
        pltpu.VMEM((SL,), jnp.float32),
        pltpu.VMEM((_NBT, BATCH), jnp.int32),
        pltpu.VMEM((_NBT, BATCH), jnp.int32),
        pltpu.VMEM((256, 16), jnp.int32),
        pltpu.VMEM((256,), jnp.int32),
        pltpu.VMEM((256,), jnp.int32),
        pltpu.VMEM((NS, 256), jnp.int32),
        pltpu.VMEM((_NBT, BATCH), jnp.int32),
        pltpu.VMEM((_NBT, BATCH), jnp.float32),
        pltpu.VMEM_SHARED((NPAD,), jnp.int32),
        pltpu.VMEM_SHARED((NPAD,), jnp.int32),
        pltpu.VMEM_SHARED((NPAD,), jnp.int32),
        pltpu.VMEM_SHARED((NPAD,), jnp.int32),
        pltpu.VMEM_SHARED((NS, 256), jnp.int32),
    ),
    compiler_params=pltpu.CompilerParams(needs_layout_passes=False),
)
def _sort_topk(score_hbm, perm_out, keep_out, sbuf, kbuf, vbuf, hist16,
               histv, offsv, hv2, oidx, kvals, k0, v0, k1, v1, hist):
  cid = lax.axis_index("c")
  sid = lax.axis_index("s")
  base = sid * SL
  iota = lax.iota(jnp.int32, 16)
  i32 = jnp.int32
  lane0 = iota == 0

  pltpu.sync_copy(score_hbm.at[pl.ds(base, SL)], sbuf)
  for i in range(SL // 16):
    s16 = sbuf[pl.ds(i * 16, 16)]
    bits = plsc.bitcast(s16, i32)
    asc = jnp.where(bits < 0, ~bits, bits ^ i32(-(2 ** 31)))
    kd = ~asc
    gi = base + i * 16 + iota
    kd = jnp.where(gi >= N, i32(-1), kd)
    b, off = i // 8, (i % 8) * 16
    kbuf[b, pl.ds(off, 16)] = kd
    vbuf[b, pl.ds(off, 16)] = gi
  for b in range(_NBT):
    pltpu.sync_copy(kbuf.at[b], k0.at[pl.ds(base + b * BATCH, BATCH)])
    pltpu.sync_copy(vbuf.at[b], v0.at[pl.ds(base + b * BATCH, BATCH)])
  plsc.subcore_barrier()

  for p in range(4):
    ksrc, vsrc = (k0, v0) if p % 2 == 0 else (k1, v1)
    kdst, vdst = (k1, v1) if p % 2 == 0 else (k0, v0)
    shift = 8 * p
    if p > 0:
      for b in range(_NBT):
        pltpu.sync_copy(ksrc.at[pl.ds(base + b * BATCH, BATCH)], kbuf.at[b])
        pltpu.sync_copy(vsrc.at[pl.ds(base + b * BATCH, BATCH)], vbuf.at[b])

    def zl(i, _):
      hist16[i, :] = jnp.zeros((16,), i32)
      return 0
    lax.fori_loop(0, 256, zl, 0)
    for i in range(SL // 16):
      kvec = kbuf[i // 8, pl.ds((i % 8) * 16, 16)]
      d = jnp.bitwise_and(lax.shift_right_logical(kvec, shift), 255)
      plsc.addupdate_scatter(hist16, [d, iota], jnp.ones((16,), i32))
    for i in range(256 // 16):
      d16 = i * 16 + iota
      acc = jnp.zeros((16,), i32)
      for l in range(16):
        acc = acc + plsc.load_gather(hist16, [d16, jnp.full((16,), l, i32)])
      histv[pl.ds(i * 16, 16)] = acc
    pltpu.sync_copy(histv, hist.at[sid])
    plsc.subcore_barrier()
    pltpu.sync_copy(hist, hv2)

    carry = jnp.zeros((16,), i32)
    for i in range(256 // 16):
      d16 = i * 16 + iota
      tot = jnp.zeros((16,), i32)
      mine = jnp.zeros((16,), i32)
      for t in range(NS):
        vals = plsc.load_gather(hv2, [jnp.full((16,), t, i32), d16])
        tot = tot + vals
        tlt = jnp.full((16,), t, i32) < jnp.full((16,), sid, i32)
        mine = mine + jnp.where(tlt, vals, jnp.zeros((16,), i32))
      excl = plsc.cumsum(tot) - tot
      offsv[pl.ds(i * 16, 16)] = carry + excl + mine
      carry = carry + jnp.sum(tot)
    plsc.subcore_barrier()

    def ploop(j, _, shift=shift):
      jb = lax.div(j, BATCH)
      jo = lax.rem(j, BATCH)
      kvec = plsc.load_gather(kbuf, [jnp.full((16,), jb, i32),
                                     jnp.full((16,), jo, i32)])
      d = jnp.bitwise_and(lax.shift_right_logical(kvec, shift), 255)
      pos = plsc.load_gather(offsv, [d])
      plsc.store_scatter(offsv, [d], pos + 1, mask=lane0)
      plsc.store_scatter(oidx, [jnp.full((16,), jb, i32),
                                jnp.full((16,), jo, i32)], pos, mask=lane0)
      return 0
    lax.fori_loop(0, SL, ploop, 0)
    for b in range(_NBT):
      pltpu.sync_copy(kbuf.at[b], kdst.at[oidx.at[b]])
      pltpu.sync_copy(vbuf.at[b], vdst.at[oidx.at[b]])
    plsc.subcore_barrier()

  for b in range(_NBT):
    pltpu.sync_copy(v0.at[pl.ds(base + b * BATCH, BATCH)], vbuf.at[b])
  for i in range(SL // 16):
    rank = base + i * 16 + iota
    kf = jnp.where(rank < K, jnp.float32(1.0), jnp.float32(0.0))
    kvals[i // 8, pl.ds((i % 8) * 16, 16)] = kf

  @pl.when(cid == 0)
  def _():
    for b in range(_NBT):
      pltpu.sync_copy(kvals.at[b], keep_out.at[vbuf.at[b]])

    @pl.when(sid < 8)
    def _():
      pltpu.sync_copy(vbuf, perm_out.at[sid])


@functools.partial(
    pl.kernel,
    out_type=jax.ShapeDtypeStruct((KPAD, H), jnp.float32),
    mesh=_mesh,
    scratch_types=(
        pltpu.VMEM((2, 80), jnp.int32),
        pltpu.VMEM((160, H), jnp.float32),
    ),
    compiler_params=pltpu.CompilerParams(use_tc_tiling_on_sc=False, needs_layout_passes=False),
)
def _gather_rows(tab_hbm, perm_hbm, out_hbm, pidx, gbuf):
  cid = lax.axis_index("c")
  sid = lax.axis_index("s")
  wid = sid * NC + cid
  pltpu.sync_copy(perm_hbm.at[wid], pidx)
  for b in range(2):
    pltpu.sync_copy(tab_hbm.at[pidx.at[b]], gbuf.at[pl.ds(b * 80, 80)])
  pltpu.sync_copy(gbuf, out_hbm.at[pl.ds(wid * 160, 160)])


def _t1_body(x_ref, w_ref, o_ref):
  xw = jnp.dot(x_ref[...], w_ref[...], preferred_element_type=jnp.float32)
  o_ref[:N, :] = xw
  o_ref[N:, :] = jnp.zeros((NPAD - N, H), jnp.float32)


def _t2_body(dp_ref, xw_ref, dinv_ref, hd_ref):
  deg = dp_ref[0] + dp_ref[1] + 1.0
  dinv = jnp.where(deg > 0, lax.rsqrt(jnp.maximum(deg, 1e-12)), 0.0)
  dinv_ref[...] = dinv
  hd_ref[...] = xw_ref[...] * dinv


def _t3_body(a1_ref, xw_ref, dinv_ref, b1_ref, h_ref, hd2_ref):
  accs = a1_ref[0] + a1_ref[1]
  dinv = dinv_ref[...]
  pre = dinv * accs + dinv * dinv * xw_ref[...] + b1_ref[...]
  h = jnp.maximum(pre, 0.0)
  zpad = jnp.zeros((NPAD - N, H), jnp.float32)
  h_ref[:N, :] = h[:N]
  h_ref[N:, :] = zpad
  hd2 = h * dinv
  hd2_ref[:N, :] = hd2[:N]
  hd2_ref[N:, :] = zpad


def _t4_body(as_ref, h_ref, dinv_ref, ws_ref, bs_ref, sc_ref):
  accs = as_ref[0] + as_ref[1]
  dinv = dinv_ref[...]
  v = dinv * accs + dinv * dinv * h_ref[...]
  pre = jnp.dot(v, ws_ref[...], preferred_element_type=jnp.float32)
  sc_ref[...] = jnp.tanh(pre + bs_ref[...])


def _t5_body(d2_ref, keep_ref, h_ref, score_ref, dinv2_ref, gp2_ref):
  deg2 = d2_ref[0] + d2_ref[1] + 1.0
  dinv2 = jnp.where(deg2 > 0, lax.rsqrt(jnp.maximum(deg2, 1e-12)), 0.0)
  dinv2_ref[...] = dinv2
  gp2_ref[...] = h_ref[...] * score_ref[...] * keep_ref[...] * dinv2


def _t6_body(a2_ref, dinv2_ref, h_ref, score_ref, pre2_ref):
  acc2 = a2_ref[0] + a2_ref[1]
  dinv2 = dinv2_ref[...]
  pre2_ref[...] = dinv2 * acc2 + dinv2 * dinv2 * (h_ref[...] * score_ref[...])


def _t7_body(hp_ref, w2_ref, b2_ref, o_ref):
  z = jnp.dot(hp_ref[...], w2_ref[...], preferred_element_type=jnp.float32)
  z = jnp.maximum(z + b2_ref[...], 0.0)
  m = jnp.max(z, axis=1, keepdims=True)
  sh = z - m
  o_ref[...] = sh - jnp.log(jnp.sum(jnp.exp(sh), axis=1, keepdims=True))


def _tc(body, out_shapes, *args):
  return pl.pallas_call(body, out_shape=out_shapes)(*args)


def kernel(x, edge_index, W1, b1, Ws, bs, W2, b2):
  f32 = jnp.float32
  row = edge_index[0].astype(jnp.int32)
  col = edge_index[1].astype(jnp.int32)
  pad = (jnp.arange(EPAD - E, dtype=jnp.int32) % PADROWS) + N
  r_p = jnp.concatenate([row, pad]).reshape(NW, NB, BATCH)
  c_p = jnp.concatenate([col, pad]).reshape(NW, NB, BATCH)

  ones_t = jnp.ones((NPAD,), f32)
  deg_parts = _agg1(ones_t, r_p, c_p).reshape(NC, NPAD, 1)

  xw = _tc(_t1_body, jax.ShapeDtypeStruct((NPAD, H), f32), x, W1)
  dinv, hd = _tc(
      _t2_body,
      (jax.ShapeDtypeStruct((NPAD, 1), f32), jax.ShapeDtypeStruct((NPAD, H), f32)),
      deg_parts, xw)

  acc1 = _agg16(hd, r_p, c_p)
  h, hd2 = _tc(
      _t3_body,
      (jax.ShapeDtypeStruct((NPAD, H), f32), jax.ShapeDtypeStruct((NPAD, H), f32)),
      acc1, xw, dinv, b1)

  accs = _agg16(hd2, r_p, c_p)
  score = _tc(_t4_body, jax.ShapeDtypeStruct((NPAD, 1), f32),
              accs, h, dinv, Ws, bs)

  perm, keep = _sort_topk(score.reshape(NPAD))
  deg2_parts = _agg1(keep, r_p, c_p).reshape(NC, NPAD, 1)
  dinv2, gp2 = _tc(
      _t5_body,
      (jax.ShapeDtypeStruct((NPAD, 1), f32), jax.ShapeDtypeStruct((NPAD, H), f32)),
      deg2_parts, keep.reshape(NPAD, 1), h, score)

  acc2 = _agg16(gp2, r_p, c_p)
  pre2 = _tc(_t6_body, jax.ShapeDtypeStruct((NPAD, H), f32),
             acc2, dinv2, h, score)

  perm3 = perm.reshape(NW, 2, 80)
  hp2 = _gather_rows(pre2, perm3)
  out = _tc(_t7_body, jax.ShapeDtypeStruct((KPAD, C), f32), hp2, W2, b2)
  return out[:K]


# speedup vs baseline: 67.3970x; 1.4261x over previous
import functools

import jax
import jax.numpy as jnp
from jax import lax
from jax.experimental import pallas as pl
from jax.experimental.pallas import tpu as pltpu
from jax.experimental.pallas import tpu_sc as plsc

N = 10000
E = 320000
D = 128
H = 16
C = 10
K = 5000

NC = 2
NS = 16
NW = NC * NS

BATCH = 128
NB = 80
EPW = NB * BATCH
EPAD = NW * EPW
PADROWS = 64
NPAD = 10240
SL = NPAD // NS
KPAD = 5120

_mesh = plsc.VectorSubcoreMesh(core_axis_name="c", subcore_axis_name="s")


def _make_agg(hdim, stage_table):
  tshape = (NPAD, hdim) if hdim > 1 else (NPAD,)
  oshape = (NC,) + tshape
  bufshape = (BATCH, hdim) if hdim > 1 else (BATCH,)
  zshape = (SL, hdim) if hdim > 1 else (SL,)
  grp = 8
  gbufshape = (grp * BATCH,) + bufshape[1:]
  scratch = [
      pltpu.VMEM((NB, BATCH), jnp.int32),
      pltpu.VMEM((NB, BATCH), jnp.int32),
      pltpu.VMEM(gbufshape, jnp.float32),
      pltpu.VMEM(gbufshape, jnp.float32),
      pltpu.VMEM(zshape, jnp.float32),
      pltpu.VMEM_SHARED(tshape, jnp.float32),
      pltpu.SemaphoreType.DMA,
      pltpu.SemaphoreType.DMA,
  ]
  if stage_table:
    scratch.append(pltpu.VMEM_SHARED(tshape, jnp.float32))

  @functools.partial(
      pl.kernel,
      out_type=jax.ShapeDtypeStruct(oshape, jnp.float32),
      mesh=_mesh,
      scratch_types=tuple(scratch),
      compiler_params=pltpu.CompilerParams(use_tc_tiling_on_sc=False, needs_layout_passes=False),
  )
  def agg(table_hbm, r_hbm, c_hbm, out_hbm, ridx, cidx, buf0, buf1, zbuf,
          acc, sem0, sem1, *maybe_tab):
    cid = lax.axis_index("c")
    sid = lax.axis_index("s")
    wid = sid * NC + cid
    sl = pl.ds(sid * SL, SL)
    pltpu.sync_copy(r_hbm.at[wid], ridx)
    pltpu.sync_copy(c_hbm.at[wid], cidx)
    if stage_table:
      tab = maybe_tab[0]
      pltpu.sync_copy(table_hbm.at[sl], tab.at[sl])
    else:
      tab = table_hbm
    if hdim > 1:
      def zb(i, _):
        zbuf[i, :] = jnp.zeros((16,), jnp.float32)
        return 0
      lax.fori_loop(0, SL, zb, 0)
    else:
      def zb(i, _):
        zbuf[pl.ds(i * 16, 16)] = jnp.zeros((16,), jnp.float32)
        return 0
      lax.fori_loop(0, SL // 16, zb, 0)
    pltpu.sync_copy(zbuf, acc.at[sl])
    plsc.subcore_barrier()

    ngrp = NB // grp
    bufs = (buf0, buf1)
    sems = (sem0, sem1)

    def fire(g):
      buf, sem = bufs[g % 2], sems[g % 2]
      for b in range(grp):
        j = g * grp + b
        pltpu.async_copy(tab.at[ridx.at[j]], buf.at[pl.ds(b * BATCH, BATCH)],
                         sem)

    def drain(g):
      buf, sem = bufs[g % 2], sems[g % 2]
      for b in range(grp):
        j = g * grp + b
        pltpu.make_async_copy(tab.at[ridx.at[j]],
                              buf.at[pl.ds(b * BATCH, BATCH)], sem).wait()

    fire(0)
    for g in range(ngrp):
      if g + 1 < ngrp:
        fire(g + 1)
      drain(g)
      buf = bufs[g % 2]
      for b in range(grp):
        j = g * grp + b
        pltpu.sync_copy(buf.at[pl.ds(b * BATCH, BATCH)], acc.at[cidx.at[j]],
                        add=True)
    plsc.subcore_barrier()
    pltpu.sync_copy(acc.at[sl], out_hbm.at[cid, sl])

  return agg


_agg16 = _make_agg(H, stage_table=False)
_agg1 = _make_agg(1, stage_table=True)


_NBT = SL // BATCH


@functools.partial(
    pl.kernel,
    out_type=(
        jax.ShapeDtypeStruct((8, _NBT, BATCH), jnp.int32),
        jax.ShapeDtypeStruct((NPAD,), jnp.float32),
    ),
    mesh=_mesh,
    scratch_types=(
        pltpu.VMEM((SL,), jnp.float32),
        pltpu.VMEM((_NBT, BATCH), jnp.int32),
        pltpu.VMEM((_NBT, BATCH), jnp.int32),
        pltpu.VMEM((256, 16), jnp.int32),
        pltpu.VMEM((256,), jnp.int32),
        pltpu.VMEM((256,), jnp.int32),
        pltpu.VMEM((NS, 256), jnp.int32),
        pltpu.VMEM((_NBT, BATCH), jnp.int32),
        pltpu.VMEM((_NBT, BATCH), jnp.float32),
        pltpu.VMEM_SHARED((NPAD,), jnp.int32),
        pltpu.VMEM_SHARED((NPAD,), jnp.int32),
        pltpu.VMEM_SHARED((NPAD,), jnp.int32),
        pltpu.VMEM_SHARED((NPAD,), jnp.int32),
        pltpu.VMEM_SHARED((NS, 256), jnp.int32),
    ),
    compiler_params=pltpu.CompilerParams(needs_layout_passes=False),
)
def _sort_topk(score_hbm, perm_out, keep_out, sbuf, kbuf, vbuf, hist16,
               histv, offsv, hv2, oidx, kvals, k0, v0, k1, v1, hist):
  cid = lax.axis_index("c")
  sid = lax.axis_index("s")
  base = sid * SL
  iota = lax.iota(jnp.int32, 16)
  i32 = jnp.int32
  lane0 = iota == 0

  pltpu.sync_copy(score_hbm.at[pl.ds(base, SL)], sbuf)
  for i in range(SL // 16):
    s16 = sbuf[pl.ds(i * 16, 16)]
    bits = plsc.bitcast(s16, i32)
    asc = jnp.where(bits < 0, ~bits, bits ^ i32(-(2 ** 31)))
    kd = ~asc
    gi = base + i * 16 + iota
    kd = jnp.where(gi >= N, i32(-1), kd)
    b, off = i // 8, (i % 8) * 16
    kbuf[b, pl.ds(off, 16)] = kd
    vbuf[b, pl.ds(off, 16)] = gi
  for b in range(_NBT):
    pltpu.sync_copy(kbuf.at[b], k0.at[pl.ds(base + b * BATCH, BATCH)])
    pltpu.sync_copy(vbuf.at[b], v0.at[pl.ds(base + b * BATCH, BATCH)])
  plsc.subcore_barrier()

  for p in range(4):
    ksrc, vsrc = (k0, v0) if p % 2 == 0 else (k1, v1)
    kdst, vdst = (k1, v1) if p % 2 == 0 else (k0, v0)
    shift = 8 * p
    if p > 0:
      for b in range(_NBT):
        pltpu.sync_copy(ksrc.at[pl.ds(base + b * BATCH, BATCH)], kbuf.at[b])
        pltpu.sync_copy(vsrc.at[pl.ds(base + b * BATCH, BATCH)], vbuf.at[b])

    def zl(i, _):
      hist16[i, :] = jnp.zeros((16,), i32)
      return 0
    lax.fori_loop(0, 256, zl, 0)
    for i in range(SL // 16):
      kvec = kbuf[i // 8, pl.ds((i % 8) * 16, 16)]
      d = jnp.bitwise_and(lax.shift_right_logical(kvec, shift), 255)
      plsc.addupdate_scatter(hist16, [d, iota], jnp.ones((16,), i32))
    for i in range(256 // 16):
      d16 = i * 16 + iota
      acc = jnp.zeros((16,), i32)
      for l in range(16):
        acc = acc + plsc.load_gather(hist16, [d16, jnp.full((16,), l, i32)])
      histv[pl.ds(i * 16, 16)] = acc
    pltpu.sync_copy(histv, hist.at[sid])
    plsc.subcore_barrier()
    pltpu.sync_copy(hist, hv2)

    carry = jnp.zeros((16,), i32)
    for i in range(256 // 16):
      d16 = i * 16 + iota
      tot = jnp.zeros((16,), i32)
      mine = jnp.zeros((16,), i32)
      for t in range(NS):
        vals = plsc.load_gather(hv2, [jnp.full((16,), t, i32), d16])
        tot = tot + vals
        tlt = jnp.full((16,), t, i32) < jnp.full((16,), sid, i32)
        mine = mine + jnp.where(tlt, vals, jnp.zeros((16,), i32))
      excl = plsc.cumsum(tot) - tot
      offsv[pl.ds(i * 16, 16)] = carry + excl + mine
      carry = carry + jnp.sum(tot)
    plsc.subcore_barrier()

    def ploop(j, _, shift=shift):
      jb = lax.div(j, BATCH)
      jo = lax.rem(j, BATCH)
      kvec = plsc.load_gather(kbuf, [jnp.full((16,), jb, i32),
                                     jnp.full((16,), jo, i32)])
      d = jnp.bitwise_and(lax.shift_right_logical(kvec, shift), 255)
      pos = plsc.load_gather(offsv, [d])
      plsc.store_scatter(offsv, [d], pos + 1, mask=lane0)
      plsc.store_scatter(oidx, [jnp.full((16,), jb, i32),
                                jnp.full((16,), jo, i32)], pos, mask=lane0)
      return 0
    lax.fori_loop(0, SL, ploop, 0, unroll=4)
    for b in range(_NBT):
      pltpu.sync_copy(kbuf.at[b], kdst.at[oidx.at[b]])
      pltpu.sync_copy(vbuf.at[b], vdst.at[oidx.at[b]])
    plsc.subcore_barrier()

  for b in range(_NBT):
    pltpu.sync_copy(v0.at[pl.ds(base + b * BATCH, BATCH)], vbuf.at[b])
  for i in range(SL // 16):
    rank = base + i * 16 + iota
    kf = jnp.where(rank < K, jnp.float32(1.0), jnp.float32(0.0))
    kvals[i // 8, pl.ds((i % 8) * 16, 16)] = kf

  @pl.when(cid == 0)
  def _():
    for b in range(_NBT):
      pltpu.sync_copy(kvals.at[b], keep_out.at[vbuf.at[b]])

    @pl.when(sid < 8)
    def _():
      pltpu.sync_copy(vbuf, perm_out.at[sid])


@functools.partial(
    pl.kernel,
    out_type=jax.ShapeDtypeStruct((KPAD, H), jnp.float32),
    mesh=_mesh,
    scratch_types=(
        pltpu.VMEM((2, 80), jnp.int32),
        pltpu.VMEM((160, H), jnp.float32),
    ),
    compiler_params=pltpu.CompilerParams(use_tc_tiling_on_sc=False, needs_layout_passes=False),
)
def _gather_rows(tab_hbm, perm_hbm, out_hbm, pidx, gbuf):
  cid = lax.axis_index("c")
  sid = lax.axis_index("s")
  wid = sid * NC + cid
  pltpu.sync_copy(perm_hbm.at[wid], pidx)
  for b in range(2):
    pltpu.sync_copy(tab_hbm.at[pidx.at[b]], gbuf.at[pl.ds(b * 80, 80)])
  pltpu.sync_copy(gbuf, out_hbm.at[pl.ds(wid * 160, 160)])


def _t1_body(x_ref, w_ref, o_ref):
  xw = jnp.dot(x_ref[...], w_ref[...], preferred_element_type=jnp.float32)
  o_ref[:N, :] = xw
  o_ref[N:, :] = jnp.zeros((NPAD - N, H), jnp.float32)


def _t2_body(dp_ref, xw_ref, dinv_ref, hd_ref):
  deg = dp_ref[0] + dp_ref[1] + 1.0
  dinv = jnp.where(deg > 0, lax.rsqrt(jnp.maximum(deg, 1e-12)), 0.0)
  dinv_ref[...] = dinv
  hd_ref[...] = xw_ref[...] * dinv


def _t3_body(a1_ref, xw_ref, dinv_ref, b1_ref, h_ref, hd2_ref):
  accs = a1_ref[0] + a1_ref[1]
  dinv = dinv_ref[...]
  pre = dinv * accs + dinv * dinv * xw_ref[...] + b1_ref[...]
  h = jnp.maximum(pre, 0.0)
  zpad = jnp.zeros((NPAD - N, H), jnp.float32)
  h_ref[:N, :] = h[:N]
  h_ref[N:, :] = zpad
  hd2 = h * dinv
  hd2_ref[:N, :] = hd2[:N]
  hd2_ref[N:, :] = zpad


def _t4_body(as_ref, h_ref, dinv_ref, ws_ref, bs_ref, sc_ref):
  accs = as_ref[0] + as_ref[1]
  dinv = dinv_ref[...]
  v = dinv * accs + dinv * dinv * h_ref[...]
  pre = jnp.dot(v, ws_ref[...], preferred_element_type=jnp.float32)
  sc_ref[...] = jnp.tanh(pre + bs_ref[...])


def _t5_body(d2_ref, keep_ref, h_ref, score_ref, dinv2_ref, gp2_ref):
  deg2 = d2_ref[0] + d2_ref[1] + 1.0
  dinv2 = jnp.where(deg2 > 0, lax.rsqrt(jnp.maximum(deg2, 1e-12)), 0.0)
  dinv2_ref[...] = dinv2
  gp2_ref[...] = h_ref[...] * score_ref[...] * keep_ref[...] * dinv2


def _t6_body(a2_ref, dinv2_ref, h_ref, score_ref, pre2_ref):
  acc2 = a2_ref[0] + a2_ref[1]
  dinv2 = dinv2_ref[...]
  pre2_ref[...] = dinv2 * acc2 + dinv2 * dinv2 * (h_ref[...] * score_ref[...])


def _t7_body(hp_ref, w2_ref, b2_ref, o_ref):
  z = jnp.dot(hp_ref[...], w2_ref[...], preferred_element_type=jnp.float32)
  z = jnp.maximum(z + b2_ref[...], 0.0)
  m = jnp.max(z, axis=1, keepdims=True)
  sh = z - m
  o_ref[...] = sh - jnp.log(jnp.sum(jnp.exp(sh), axis=1, keepdims=True))


def _tc(body, out_shapes, *args):
  return pl.pallas_call(body, out_shape=out_shapes)(*args)


def kernel(x, edge_index, W1, b1, Ws, bs, W2, b2):
  f32 = jnp.float32
  row = edge_index[0].astype(jnp.int32)
  col = edge_index[1].astype(jnp.int32)
  pad = (jnp.arange(EPAD - E, dtype=jnp.int32) % PADROWS) + N
  r_p = jnp.concatenate([row, pad]).reshape(NW, NB, BATCH)
  c_p = jnp.concatenate([col, pad]).reshape(NW, NB, BATCH)

  ones_t = jnp.ones((NPAD,), f32)
  deg_parts = _agg1(ones_t, r_p, c_p).reshape(NC, NPAD, 1)

  xw = _tc(_t1_body, jax.ShapeDtypeStruct((NPAD, H), f32), x, W1)
  dinv, hd = _tc(
      _t2_body,
      (jax.ShapeDtypeStruct((NPAD, 1), f32), jax.ShapeDtypeStruct((NPAD, H), f32)),
      deg_parts, xw)

  acc1 = _agg16(hd, r_p, c_p)
  h, hd2 = _tc(
      _t3_body,
      (jax.ShapeDtypeStruct((NPAD, H), f32), jax.ShapeDtypeStruct((NPAD, H), f32)),
      acc1, xw, dinv, b1)

  accs = _agg16(hd2, r_p, c_p)
  score = _tc(_t4_body, jax.ShapeDtypeStruct((NPAD, 1), f32),
              accs, h, dinv, Ws, bs)

  perm, keep = _sort_topk(score.reshape(NPAD))
  deg2_parts = _agg1(keep, r_p, c_p).reshape(NC, NPAD, 1)
  dinv2, gp2 = _tc(
      _t5_body,
      (jax.ShapeDtypeStruct((NPAD, 1), f32), jax.ShapeDtypeStruct((NPAD, H), f32)),
      deg2_parts, keep.reshape(NPAD, 1), h, score)

  acc2 = _agg16(gp2, r_p, c_p)
  pre2 = _tc(_t6_body, jax.ShapeDtypeStruct((NPAD, H), f32),
             acc2, dinv2, h, score)

  perm3 = perm.reshape(NW, 2, 80)
  hp2 = _gather_rows(pre2, perm3)
  out = _tc(_t7_body, jax.ShapeDtypeStruct((KPAD, C), f32), hp2, W2, b2)
  return out[:K]


# speedup vs baseline: 68.6201x; 1.0181x over previous
import functools

import jax
import jax.numpy as jnp
from jax import lax
from jax.experimental import pallas as pl
from jax.experimental.pallas import tpu as pltpu
from jax.experimental.pallas import tpu_sc as plsc

N = 10000
E = 320000
D = 128
H = 16
C = 10
K = 5000

NC = 2
NS = 16
NW = NC * NS

BATCH = 128
NB = 80
EPW = NB * BATCH
EPAD = NW * EPW
PADROWS = 64
NPAD = 10240
SL = NPAD // NS
KPAD = 5120

_mesh = plsc.VectorSubcoreMesh(core_axis_name="c", subcore_axis_name="s")


def _make_agg(hdim, stage_table):
  tshape = (NPAD, hdim) if hdim > 1 else (NPAD,)
  oshape = (NC,) + tshape
  bufshape = (BATCH, hdim) if hdim > 1 else (BATCH,)
  zshape = (SL, hdim) if hdim > 1 else (SL,)
  grp = 8
  gbufshape = (grp * BATCH,) + bufshape[1:]
  scratch = [
      pltpu.VMEM((NB, BATCH), jnp.int32),
      pltpu.VMEM((NB, BATCH), jnp.int32),
      pltpu.VMEM(gbufshape, jnp.float32),
      pltpu.VMEM(gbufshape, jnp.float32),
      pltpu.VMEM(zshape, jnp.float32),
      pltpu.VMEM_SHARED(tshape, jnp.float32),
      pltpu.SemaphoreType.DMA,
      pltpu.SemaphoreType.DMA,
  ]
  if stage_table:
    scratch.append(pltpu.VMEM_SHARED(tshape, jnp.float32))

  @functools.partial(
      pl.kernel,
      out_type=jax.ShapeDtypeStruct(oshape, jnp.float32),
      mesh=_mesh,
      scratch_types=tuple(scratch),
      compiler_params=pltpu.CompilerParams(use_tc_tiling_on_sc=False, needs_layout_passes=False),
  )
  def agg(table_hbm, r_hbm, c_hbm, out_hbm, ridx, cidx, buf0, buf1, zbuf,
          acc, sem0, sem1, *maybe_tab):
    cid = lax.axis_index("c")
    sid = lax.axis_index("s")
    wid = sid * NC + cid
    sl = pl.ds(sid * SL, SL)
    pltpu.sync_copy(r_hbm.at[wid], ridx)
    pltpu.sync_copy(c_hbm.at[wid], cidx)
    if stage_table:
      tab = maybe_tab[0]
      pltpu.sync_copy(table_hbm.at[sl], tab.at[sl])
    else:
      tab = table_hbm
    if hdim > 1:
      def zb(i, _):
        zbuf[i, :] = jnp.zeros((16,), jnp.float32)
        return 0
      lax.fori_loop(0, SL, zb, 0)
    else:
      def zb(i, _):
        zbuf[pl.ds(i * 16, 16)] = jnp.zeros((16,), jnp.float32)
        return 0
      lax.fori_loop(0, SL // 16, zb, 0)
    pltpu.sync_copy(zbuf, acc.at[sl])
    plsc.subcore_barrier()

    ngrp = NB // grp
    bufs = (buf0, buf1)
    sems = (sem0, sem1)

    def fire(g):
      buf, sem = bufs[g % 2], sems[g % 2]
      for b in range(grp):
        j = g * grp + b
        pltpu.async_copy(tab.at[ridx.at[j]], buf.at[pl.ds(b * BATCH, BATCH)],
                         sem)

    def drain(g):
      buf, sem = bufs[g % 2], sems[g % 2]
      for b in range(grp):
        j = g * grp + b
        pltpu.make_async_copy(tab.at[ridx.at[j]],
                              buf.at[pl.ds(b * BATCH, BATCH)], sem).wait()

    fire(0)
    for g in range(ngrp):
      if g + 1 < ngrp:
        fire(g + 1)
      drain(g)
      buf = bufs[g % 2]
      for b in range(grp):
        j = g * grp + b
        pltpu.sync_copy(buf.at[pl.ds(b * BATCH, BATCH)], acc.at[cidx.at[j]],
                        add=True)
    plsc.subcore_barrier()
    pltpu.sync_copy(acc.at[sl], out_hbm.at[cid, sl])

  return agg


_agg16 = _make_agg(H, stage_table=False)
_agg1 = _make_agg(1, stage_table=True)


_NBT = SL // BATCH


@functools.partial(
    pl.kernel,
    out_type=(
        jax.ShapeDtypeStruct((8, _NBT, BATCH), jnp.int32),
        jax.ShapeDtypeStruct((NPAD,), jnp.float32),
    ),
    mesh=_mesh,
    scratch_types=(
        pltpu.VMEM((SL,), jnp.float32),
        pltpu.VMEM((_NBT, BATCH), jnp.int32),
        pltpu.VMEM((_NBT, BATCH), jnp.int32),
        pltpu.VMEM((256, 16), jnp.int32),
        pltpu.VMEM((256,), jnp.int32),
        pltpu.VMEM((256,), jnp.int32),
        pltpu.VMEM((NS, 256), jnp.int32),
        pltpu.VMEM((_NBT, BATCH), jnp.int32),
        pltpu.VMEM((_NBT, BATCH), jnp.float32),
        pltpu.VMEM_SHARED((NPAD,), jnp.int32),
        pltpu.VMEM_SHARED((NPAD,), jnp.int32),
        pltpu.VMEM_SHARED((NPAD,), jnp.int32),
        pltpu.VMEM_SHARED((NPAD,), jnp.int32),
        pltpu.VMEM_SHARED((NS, 256), jnp.int32),
    ),
    compiler_params=pltpu.CompilerParams(needs_layout_passes=False),
)
def _sort_topk(score_hbm, perm_out, keep_out,
               sbuf, kbuf, vbuf, hist16, histv, offsv, hv2, oidx,
               kvals, k0, v0, k1, v1, hist):
  cid = lax.axis_index("c")
  sid = lax.axis_index("s")
  base = sid * SL
  iota = lax.iota(jnp.int32, 16)
  i32 = jnp.int32
  lane0 = iota == 0

  pltpu.sync_copy(score_hbm.at[pl.ds(base, SL)], sbuf)
  for i in range(SL // 16):
    s16 = sbuf[pl.ds(i * 16, 16)]
    bits = plsc.bitcast(s16, i32)
    asc = jnp.where(bits < 0, ~bits, bits ^ i32(-(2 ** 31)))
    kd = ~asc
    gi = base + i * 16 + iota
    kd = jnp.where(gi >= N, i32(-1), kd)
    b, off = i // 8, (i % 8) * 16
    kbuf[b, pl.ds(off, 16)] = kd
    vbuf[b, pl.ds(off, 16)] = gi
  for b in range(_NBT):
    pltpu.sync_copy(kbuf.at[b], k0.at[pl.ds(base + b * BATCH, BATCH)])
    pltpu.sync_copy(vbuf.at[b], v0.at[pl.ds(base + b * BATCH, BATCH)])
  plsc.subcore_barrier()

  for p in range(4):
    ksrc, vsrc = (k0, v0) if p % 2 == 0 else (k1, v1)
    kdst, vdst = (k1, v1) if p % 2 == 0 else (k0, v0)
    shift = 8 * p
    if p > 0:
      for b in range(_NBT):
        pltpu.sync_copy(ksrc.at[pl.ds(base + b * BATCH, BATCH)], kbuf.at[b])
        pltpu.sync_copy(vsrc.at[pl.ds(base + b * BATCH, BATCH)], vbuf.at[b])

    def zl(i, _):
      hist16[i, :] = jnp.zeros((16,), i32)
      return 0
    lax.fori_loop(0, 256, zl, 0)
    for i in range(SL // 16):
      kvec = kbuf[i // 8, pl.ds((i % 8) * 16, 16)]
      d = jnp.bitwise_and(lax.shift_right_logical(kvec, shift), 255)
      plsc.addupdate_scatter(hist16, [d, iota], jnp.ones((16,), i32))
    for i in range(256 // 16):
      d16 = i * 16 + iota
      acc = jnp.zeros((16,), i32)
      for l in range(16):
        acc = acc + plsc.load_gather(hist16, [d16, jnp.full((16,), l, i32)])
      histv[pl.ds(i * 16, 16)] = acc
    pltpu.sync_copy(histv, hist.at[sid])
    plsc.subcore_barrier()
    pltpu.sync_copy(hist, hv2)

    carry = jnp.zeros((16,), i32)
    for i in range(256 // 16):
      d16 = i * 16 + iota
      tot = jnp.zeros((16,), i32)
      mine = jnp.zeros((16,), i32)
      for t in range(NS):
        vals = plsc.load_gather(hv2, [jnp.full((16,), t, i32), d16])
        tot = tot + vals
        tlt = jnp.full((16,), t, i32) < jnp.full((16,), sid, i32)
        mine = mine + jnp.where(tlt, vals, jnp.zeros((16,), i32))
      excl = plsc.cumsum(tot) - tot
      offsv[pl.ds(i * 16, 16)] = carry + excl + mine
      carry = carry + jnp.sum(tot)
    plsc.subcore_barrier()

    def ploop(j, _, shift=shift):
      jb = lax.div(j, BATCH)
      jo = lax.rem(j, BATCH)
      kvec = plsc.load_gather(kbuf, [jnp.full((16,), jb, i32),
                                     jnp.full((16,), jo, i32)])
      d = jnp.bitwise_and(lax.shift_right_logical(kvec, shift), 255)
      pos = plsc.load_gather(offsv, [d])
      plsc.store_scatter(offsv, [d], pos + 1, mask=lane0)
      plsc.store_scatter(oidx, [jnp.full((16,), jb, i32),
                                jnp.full((16,), jo, i32)], pos, mask=lane0)
      return 0
    lax.fori_loop(0, SL, ploop, 0, unroll=4)
    for b in range(_NBT):
      pltpu.sync_copy(kbuf.at[b], kdst.at[oidx.at[b]])
      pltpu.sync_copy(vbuf.at[b], vdst.at[oidx.at[b]])
    plsc.subcore_barrier()

  for b in range(_NBT):
    pltpu.sync_copy(v0.at[pl.ds(base + b * BATCH, BATCH)], vbuf.at[b])
  for i in range(SL // 16):
    rank = base + i * 16 + iota
    kf = jnp.where(rank < K, jnp.float32(1.0), jnp.float32(0.0))
    kvals[i // 8, pl.ds((i % 8) * 16, 16)] = kf

  @pl.when(cid == 0)
  def _():
    for b in range(_NBT):
      pltpu.sync_copy(kvals.at[b], keep_out.at[vbuf.at[b]])

    @pl.when(sid < 8)
    def _():
      pltpu.sync_copy(vbuf, perm_out.at[sid])


@functools.partial(
    pl.kernel,
    out_type=jax.ShapeDtypeStruct((KPAD, H), jnp.float32),
    mesh=_mesh,
    scratch_types=(
        pltpu.VMEM((2, 80), jnp.int32),
        pltpu.VMEM((160, H), jnp.float32),
    ),
    compiler_params=pltpu.CompilerParams(use_tc_tiling_on_sc=False, needs_layout_passes=False),
)
def _gather_rows(tab_hbm, perm_hbm, out_hbm, pidx, gbuf):
  cid = lax.axis_index("c")
  sid = lax.axis_index("s")
  wid = sid * NC + cid
  pltpu.sync_copy(perm_hbm.at[wid], pidx)
  for b in range(2):
    pltpu.sync_copy(tab_hbm.at[pidx.at[b]], gbuf.at[pl.ds(b * 80, 80)])
  pltpu.sync_copy(gbuf, out_hbm.at[pl.ds(wid * 160, 160)])


def _t12_body(x_ref, w_ref, dp_ref, dinv_ref, hd_ref):
  xw = jnp.dot(x_ref[...], w_ref[...], preferred_element_type=jnp.float32)
  deg = dp_ref[0] + dp_ref[1] + 1.0
  dinv = jnp.where(deg > 0, lax.rsqrt(jnp.maximum(deg, 1e-12)), 0.0)
  dinv_ref[...] = dinv
  hd_ref[:N, :] = xw * dinv[:N]
  hd_ref[N:, :] = jnp.zeros((NPAD - N, H), jnp.float32)


def _t3_body(a1_ref, hd_ref, dinv_ref, b1_ref, h_ref, hd2_ref):
  accs = a1_ref[0] + a1_ref[1]
  dinv = dinv_ref[...]
  pre = dinv * accs + dinv * hd_ref[...] + b1_ref[...]
  h = jnp.maximum(pre, 0.0)
  zpad = jnp.zeros((NPAD - N, H), jnp.float32)
  h_ref[:N, :] = h[:N]
  h_ref[N:, :] = zpad
  hd2 = h * dinv
  hd2_ref[:N, :] = hd2[:N]
  hd2_ref[N:, :] = zpad


def _t4_body(as_ref, h_ref, dinv_ref, ws_ref, bs_ref, sc_ref):
  accs = as_ref[0] + as_ref[1]
  dinv = dinv_ref[...]
  v = dinv * accs + dinv * dinv * h_ref[...]
  pre = jnp.dot(v, ws_ref[...], preferred_element_type=jnp.float32)
  sc_ref[...] = jnp.tanh(pre + bs_ref[...])


def _t5_body(d2_ref, keep_ref, h_ref, score_ref, dinv2_ref, gp2_ref):
  deg2 = d2_ref[0] + d2_ref[1] + 1.0
  dinv2 = jnp.where(deg2 > 0, lax.rsqrt(jnp.maximum(deg2, 1e-12)), 0.0)
  dinv2_ref[...] = dinv2
  gp2_ref[...] = h_ref[...] * score_ref[...] * keep_ref[...] * dinv2


def _t6_body(a2_ref, dinv2_ref, gp2_ref, pre2_ref):
  acc2 = a2_ref[0] + a2_ref[1]
  dinv2 = dinv2_ref[...]
  pre2_ref[...] = dinv2 * acc2 + dinv2 * gp2_ref[...]


def _t7_body(hp_ref, w2_ref, b2_ref, o_ref):
  z = jnp.dot(hp_ref[...], w2_ref[...], preferred_element_type=jnp.float32)
  z = jnp.maximum(z + b2_ref[...], 0.0)
  m = jnp.max(z, axis=1, keepdims=True)
  sh = z - m
  o_ref[...] = sh - jnp.log(jnp.sum(jnp.exp(sh), axis=1, keepdims=True))


def _tc(body, out_shapes, *args):
  return pl.pallas_call(body, out_shape=out_shapes)(*args)


def kernel(x, edge_index, W1, b1, Ws, bs, W2, b2):
  f32 = jnp.float32
  row = edge_index[0].astype(jnp.int32)
  col = edge_index[1].astype(jnp.int32)
  pad = (jnp.arange(EPAD - E, dtype=jnp.int32) % PADROWS) + N
  r_p = jnp.concatenate([row, pad]).reshape(NW, NB, BATCH)
  c_p = jnp.concatenate([col, pad]).reshape(NW, NB, BATCH)

  ones_t = jnp.ones((NPAD,), f32)
  deg_parts = _agg1(ones_t, r_p, c_p).reshape(NC, NPAD, 1)

  dinv, hd = _tc(
      _t12_body,
      (jax.ShapeDtypeStruct((NPAD, 1), f32), jax.ShapeDtypeStruct((NPAD, H), f32)),
      x, W1, deg_parts)

  acc1 = _agg16(hd, r_p, c_p)
  h, hd2 = _tc(
      _t3_body,
      (jax.ShapeDtypeStruct((NPAD, H), f32), jax.ShapeDtypeStruct((NPAD, H), f32)),
      acc1, hd, dinv, b1)

  accs = _agg16(hd2, r_p, c_p)
  score = _tc(_t4_body, jax.ShapeDtypeStruct((NPAD, 1), f32),
              accs, h, dinv, Ws, bs)

  perm, keep = _sort_topk(score.reshape(NPAD))
  deg2_parts = _agg1(keep, r_p, c_p).reshape(NC, NPAD, 1)
  dinv2, gp2 = _tc(
      _t5_body,
      (jax.ShapeDtypeStruct((NPAD, 1), f32), jax.ShapeDtypeStruct((NPAD, H), f32)),
      deg2_parts, keep.reshape(NPAD, 1), h, score)

  acc2 = _agg16(gp2, r_p, c_p)
  pre2 = _tc(_t6_body, jax.ShapeDtypeStruct((NPAD, H), f32),
             acc2, dinv2, gp2)

  perm3 = perm.reshape(NW, 2, 80)
  hp2 = _gather_rows(pre2, perm3)
  out = _tc(_t7_body, jax.ShapeDtypeStruct((KPAD, C), f32), hp2, W2, b2)
  return out[:K]


# speedup vs baseline: 69.4309x; 1.0118x over previous
import functools

import jax
import jax.numpy as jnp
from jax import lax
from jax.experimental import pallas as pl
from jax.experimental.pallas import tpu as pltpu
from jax.experimental.pallas import tpu_sc as plsc

N = 10000
E = 320000
D = 128
H = 16
C = 10
K = 5000

NC = 2
NS = 16
NW = NC * NS

BATCH = 128
NB = 80
EPW = NB * BATCH
EPAD = NW * EPW
PADROWS = 64
NPAD = 10240
SL = NPAD // NS
KPAD = 5120

_mesh = plsc.VectorSubcoreMesh(core_axis_name="c", subcore_axis_name="s")


def _make_agg(hdim, stage_table):
  tshape = (NPAD, hdim) if hdim > 1 else (NPAD,)
  oshape = (NC,) + tshape
  bufshape = (BATCH, hdim) if hdim > 1 else (BATCH,)
  zshape = (SL, hdim) if hdim > 1 else (SL,)
  grp = 8
  gbufshape = (grp * BATCH,) + bufshape[1:]
  scratch = [
      pltpu.VMEM((NB, BATCH), jnp.int32),
      pltpu.VMEM((NB, BATCH), jnp.int32),
      pltpu.VMEM(gbufshape, jnp.float32),
      pltpu.VMEM(gbufshape, jnp.float32),
      pltpu.VMEM(zshape, jnp.float32),
      pltpu.VMEM_SHARED(tshape, jnp.float32),
      pltpu.SemaphoreType.DMA,
      pltpu.SemaphoreType.DMA,
      pltpu.SemaphoreType.DMA,
      pltpu.SemaphoreType.DMA,
  ]
  if stage_table:
    scratch.append(pltpu.VMEM_SHARED(tshape, jnp.float32))

  @functools.partial(
      pl.kernel,
      out_type=jax.ShapeDtypeStruct(oshape, jnp.float32),
      mesh=_mesh,
      scratch_types=tuple(scratch),
      compiler_params=pltpu.CompilerParams(use_tc_tiling_on_sc=False, needs_layout_passes=False),
  )
  def agg(table_hbm, r_hbm, c_hbm, out_hbm, ridx, cidx, buf0, buf1, zbuf,
          acc, sem0, sem1, ssem0, ssem1, *maybe_tab):
    cid = lax.axis_index("c")
    sid = lax.axis_index("s")
    wid = sid * NC + cid
    sl = pl.ds(sid * SL, SL)
    pltpu.sync_copy(r_hbm.at[wid], ridx)
    pltpu.sync_copy(c_hbm.at[wid], cidx)
    if stage_table:
      tab = maybe_tab[0]
      pltpu.sync_copy(table_hbm.at[sl], tab.at[sl])
    else:
      tab = table_hbm
    if hdim > 1:
      def zb(i, _):
        zbuf[i, :] = jnp.zeros((16,), jnp.float32)
        return 0
      lax.fori_loop(0, SL, zb, 0)
    else:
      def zb(i, _):
        zbuf[pl.ds(i * 16, 16)] = jnp.zeros((16,), jnp.float32)
        return 0
      lax.fori_loop(0, SL // 16, zb, 0)
    pltpu.sync_copy(zbuf, acc.at[sl])
    plsc.subcore_barrier()

    ngrp = NB // grp
    bufs = (buf0, buf1)
    sems = (sem0, sem1)
    ssems = (ssem0, ssem1)

    def fire(g):
      buf, sem = bufs[g % 2], sems[g % 2]
      for b in range(grp):
        j = g * grp + b
        pltpu.async_copy(tab.at[ridx.at[j]], buf.at[pl.ds(b * BATCH, BATCH)],
                         sem)

    def drain(g):
      buf, sem = bufs[g % 2], sems[g % 2]
      for b in range(grp):
        j = g * grp + b
        pltpu.make_async_copy(tab.at[ridx.at[j]],
                              buf.at[pl.ds(b * BATCH, BATCH)], sem).wait()

    def fire_sc(g):
      buf, sem = bufs[g % 2], ssems[g % 2]
      for b in range(grp):
        j = g * grp + b
        pltpu.async_copy(buf.at[pl.ds(b * BATCH, BATCH)], acc.at[cidx.at[j]],
                         sem, add=True)

    def drain_sc(g):
      buf, sem = bufs[g % 2], ssems[g % 2]
      for b in range(grp):
        j = g * grp + b
        pltpu.make_async_copy(buf.at[pl.ds(b * BATCH, BATCH)],
                              acc.at[cidx.at[j]], sem).wait()

    fire(0)
    for g in range(ngrp):
      if g >= 1:
        drain_sc(g - 1)
      if g + 1 < ngrp:
        fire(g + 1)
      drain(g)
      fire_sc(g)
    drain_sc(ngrp - 1)
    plsc.subcore_barrier()
    pltpu.sync_copy(acc.at[sl], out_hbm.at[cid, sl])

  return agg


_agg16 = _make_agg(H, stage_table=False)
_agg1 = _make_agg(1, stage_table=True)


_NBT = SL // BATCH


@functools.partial(
    pl.kernel,
    out_type=(
        jax.ShapeDtypeStruct((8, _NBT, BATCH), jnp.int32),
        jax.ShapeDtypeStruct((NPAD,), jnp.float32),
    ),
    mesh=_mesh,
    scratch_types=(
        pltpu.VMEM((SL,), jnp.float32),
        pltpu.VMEM((_NBT, BATCH), jnp.int32),
        pltpu.VMEM((_NBT, BATCH), jnp.int32),
        pltpu.VMEM((256, 16), jnp.int32),
        pltpu.VMEM((256,), jnp.int32),
        pltpu.VMEM((256,), jnp.int32),
        pltpu.VMEM((NS, 256), jnp.int32),
        pltpu.VMEM((_NBT, BATCH), jnp.int32),
        pltpu.VMEM((_NBT, BATCH), jnp.float32),
        pltpu.VMEM_SHARED((NPAD,), jnp.int32),
        pltpu.VMEM_SHARED((NPAD,), jnp.int32),
        pltpu.VMEM_SHARED((NPAD,), jnp.int32),
        pltpu.VMEM_SHARED((NPAD,), jnp.int32),
        pltpu.VMEM_SHARED((NS, 256), jnp.int32),
    ),
    compiler_params=pltpu.CompilerParams(needs_layout_passes=False),
)
def _sort_topk(score_hbm, perm_out, keep_out,
               sbuf, kbuf, vbuf, hist16, histv, offsv, hv2, oidx,
               kvals, k0, v0, k1, v1, hist):
  cid = lax.axis_index("c")
  sid = lax.axis_index("s")
  base = sid * SL
  iota = lax.iota(jnp.int32, 16)
  i32 = jnp.int32
  lane0 = iota == 0

  pltpu.sync_copy(score_hbm.at[pl.ds(base, SL)], sbuf)
  for i in range(SL // 16):
    s16 = sbuf[pl.ds(i * 16, 16)]
    bits = plsc.bitcast(s16, i32)
    asc = jnp.where(bits < 0, ~bits, bits ^ i32(-(2 ** 31)))
    kd = ~asc
    gi = base + i * 16 + iota
    kd = jnp.where(gi >= N, i32(-1), kd)
    b, off = i // 8, (i % 8) * 16
    kbuf[b, pl.ds(off, 16)] = kd
    vbuf[b, pl.ds(off, 16)] = gi
  for b in range(_NBT):
    pltpu.sync_copy(kbuf.at[b], k0.at[pl.ds(base + b * BATCH, BATCH)])
    pltpu.sync_copy(vbuf.at[b], v0.at[pl.ds(base + b * BATCH, BATCH)])
  plsc.subcore_barrier()

  for p in range(4):
    ksrc, vsrc = (k0, v0) if p % 2 == 0 else (k1, v1)
    kdst, vdst = (k1, v1) if p % 2 == 0 else (k0, v0)
    shift = 8 * p
    if p > 0:
      for b in range(_NBT):
        pltpu.sync_copy(ksrc.at[pl.ds(base + b * BATCH, BATCH)], kbuf.at[b])
        pltpu.sync_copy(vsrc.at[pl.ds(base + b * BATCH, BATCH)], vbuf.at[b])

    def zl(i, _):
      hist16[i, :] = jnp.zeros((16,), i32)
      return 0
    lax.fori_loop(0, 256, zl, 0)
    for i in range(SL // 16):
      kvec = kbuf[i // 8, pl.ds((i % 8) * 16, 16)]
      d = jnp.bitwise_and(lax.shift_right_logical(kvec, shift), 255)
      plsc.addupdate_scatter(hist16, [d, iota], jnp.ones((16,), i32))
    for i in range(256 // 16):
      d16 = i * 16 + iota
      acc = jnp.zeros((16,), i32)
      for l in range(16):
        acc = acc + plsc.load_gather(hist16, [d16, jnp.full((16,), l, i32)])
      histv[pl.ds(i * 16, 16)] = acc
    pltpu.sync_copy(histv, hist.at[sid])
    plsc.subcore_barrier()
    pltpu.sync_copy(hist, hv2)

    carry = jnp.zeros((16,), i32)
    for i in range(256 // 16):
      d16 = i * 16 + iota
      tot = jnp.zeros((16,), i32)
      mine = jnp.zeros((16,), i32)
      for t in range(NS):
        vals = plsc.load_gather(hv2, [jnp.full((16,), t, i32), d16])
        tot = tot + vals
        tlt = jnp.full((16,), t, i32) < jnp.full((16,), sid, i32)
        mine = mine + jnp.where(tlt, vals, jnp.zeros((16,), i32))
      excl = plsc.cumsum(tot) - tot
      offsv[pl.ds(i * 16, 16)] = carry + excl + mine
      carry = carry + jnp.sum(tot)
    plsc.subcore_barrier()

    def ploop(j, _, shift=shift):
      jb = lax.div(j, BATCH)
      jo = lax.rem(j, BATCH)
      kvec = plsc.load_gather(kbuf, [jnp.full((16,), jb, i32),
                                     jnp.full((16,), jo, i32)])
      d = jnp.bitwise_and(lax.shift_right_logical(kvec, shift), 255)
      pos = plsc.load_gather(offsv, [d])
      plsc.store_scatter(offsv, [d], pos + 1, mask=lane0)
      plsc.store_scatter(oidx, [jnp.full((16,), jb, i32),
                                jnp.full((16,), jo, i32)], pos, mask=lane0)
      return 0
    lax.fori_loop(0, SL, ploop, 0, unroll=4)
    for b in range(_NBT):
      pltpu.sync_copy(kbuf.at[b], kdst.at[oidx.at[b]])
      pltpu.sync_copy(vbuf.at[b], vdst.at[oidx.at[b]])
    plsc.subcore_barrier()

  for b in range(_NBT):
    pltpu.sync_copy(v0.at[pl.ds(base + b * BATCH, BATCH)], vbuf.at[b])
  for i in range(SL // 16):
    rank = base + i * 16 + iota
    kf = jnp.where(rank < K, jnp.float32(1.0), jnp.float32(0.0))
    kvals[i // 8, pl.ds((i % 8) * 16, 16)] = kf

  @pl.when(cid == 0)
  def _():
    for b in range(_NBT):
      pltpu.sync_copy(kvals.at[b], keep_out.at[vbuf.at[b]])

    @pl.when(sid < 8)
    def _():
      pltpu.sync_copy(vbuf, perm_out.at[sid])


@functools.partial(
    pl.kernel,
    out_type=jax.ShapeDtypeStruct((KPAD, H), jnp.float32),
    mesh=_mesh,
    scratch_types=(
        pltpu.VMEM((2, 80), jnp.int32),
        pltpu.VMEM((160, H), jnp.float32),
    ),
    compiler_params=pltpu.CompilerParams(use_tc_tiling_on_sc=False, needs_layout_passes=False),
)
def _gather_rows(tab_hbm, perm_hbm, out_hbm, pidx, gbuf):
  cid = lax.axis_index("c")
  sid = lax.axis_index("s")
  wid = sid * NC + cid
  pltpu.sync_copy(perm_hbm.at[wid], pidx)
  for b in range(2):
    pltpu.sync_copy(tab_hbm.at[pidx.at[b]], gbuf.at[pl.ds(b * 80, 80)])
  pltpu.sync_copy(gbuf, out_hbm.at[pl.ds(wid * 160, 160)])


def _t12_body(x_ref, w_ref, dp_ref, dinv_ref, hd_ref):
  xw = jnp.dot(x_ref[...], w_ref[...], preferred_element_type=jnp.float32)
  deg = dp_ref[0] + dp_ref[1] + 1.0
  dinv = jnp.where(deg > 0, lax.rsqrt(jnp.maximum(deg, 1e-12)), 0.0)
  dinv_ref[...] = dinv
  hd_ref[:N, :] = xw * dinv[:N]
  hd_ref[N:, :] = jnp.zeros((NPAD - N, H), jnp.float32)


def _t3_body(a1_ref, hd_ref, dinv_ref, b1_ref, h_ref, hd2_ref):
  accs = a1_ref[0] + a1_ref[1]
  dinv = dinv_ref[...]
  pre = dinv * accs + dinv * hd_ref[...] + b1_ref[...]
  h = jnp.maximum(pre, 0.0)
  zpad = jnp.zeros((NPAD - N, H), jnp.float32)
  h_ref[:N, :] = h[:N]
  h_ref[N:, :] = zpad
  hd2 = h * dinv
  hd2_ref[:N, :] = hd2[:N]
  hd2_ref[N:, :] = zpad


def _t4_body(as_ref, h_ref, dinv_ref, ws_ref, bs_ref, sc_ref):
  accs = as_ref[0] + as_ref[1]
  dinv = dinv_ref[...]
  v = dinv * accs + dinv * dinv * h_ref[...]
  pre = jnp.dot(v, ws_ref[...], preferred_element_type=jnp.float32)
  sc_ref[...] = jnp.tanh(pre + bs_ref[...])


def _t5_body(d2_ref, keep_ref, h_ref, score_ref, dinv2_ref, gp2_ref):
  deg2 = d2_ref[0] + d2_ref[1] + 1.0
  dinv2 = jnp.where(deg2 > 0, lax.rsqrt(jnp.maximum(deg2, 1e-12)), 0.0)
  dinv2_ref[...] = dinv2
  gp2_ref[...] = h_ref[...] * score_ref[...] * keep_ref[...] * dinv2


def _t6_body(a2_ref, dinv2_ref, gp2_ref, pre2_ref):
  acc2 = a2_ref[0] + a2_ref[1]
  dinv2 = dinv2_ref[...]
  pre2_ref[...] = dinv2 * acc2 + dinv2 * gp2_ref[...]


def _t7_body(hp_ref, w2_ref, b2_ref, o_ref):
  z = jnp.dot(hp_ref[...], w2_ref[...], preferred_element_type=jnp.float32)
  z = jnp.maximum(z + b2_ref[...], 0.0)
  m = jnp.max(z, axis=1, keepdims=True)
  sh = z - m
  o_ref[...] = sh - jnp.log(jnp.sum(jnp.exp(sh), axis=1, keepdims=True))


def _tc(body, out_shapes, *args):
  return pl.pallas_call(body, out_shape=out_shapes)(*args)


def kernel(x, edge_index, W1, b1, Ws, bs, W2, b2):
  f32 = jnp.float32
  row = edge_index[0].astype(jnp.int32)
  col = edge_index[1].astype(jnp.int32)
  pad = (jnp.arange(EPAD - E, dtype=jnp.int32) % PADROWS) + N
  r_p = jnp.concatenate([row, pad]).reshape(NW, NB, BATCH)
  c_p = jnp.concatenate([col, pad]).reshape(NW, NB, BATCH)

  ones_t = jnp.ones((NPAD,), f32)
  deg_parts = _agg1(ones_t, r_p, c_p).reshape(NC, NPAD, 1)

  dinv, hd = _tc(
      _t12_body,
      (jax.ShapeDtypeStruct((NPAD, 1), f32), jax.ShapeDtypeStruct((NPAD, H), f32)),
      x, W1, deg_parts)

  acc1 = _agg16(hd, r_p, c_p)
  h, hd2 = _tc(
      _t3_body,
      (jax.ShapeDtypeStruct((NPAD, H), f32), jax.ShapeDtypeStruct((NPAD, H), f32)),
      acc1, hd, dinv, b1)

  accs = _agg16(hd2, r_p, c_p)
  score = _tc(_t4_body, jax.ShapeDtypeStruct((NPAD, 1), f32),
              accs, h, dinv, Ws, bs)

  perm, keep = _sort_topk(score.reshape(NPAD))
  deg2_parts = _agg1(keep, r_p, c_p).reshape(NC, NPAD, 1)
  dinv2, gp2 = _tc(
      _t5_body,
      (jax.ShapeDtypeStruct((NPAD, 1), f32), jax.ShapeDtypeStruct((NPAD, H), f32)),
      deg2_parts, keep.reshape(NPAD, 1), h, score)

  acc2 = _agg16(gp2, r_p, c_p)
  pre2 = _tc(_t6_body, jax.ShapeDtypeStruct((NPAD, H), f32),
             acc2, dinv2, gp2)

  perm3 = perm.reshape(NW, 2, 80)
  hp2 = _gather_rows(pre2, perm3)
  out = _tc(_t7_body, jax.ShapeDtypeStruct((KPAD, C), f32), hp2, W2, b2)
  return out[:K]
